# baseline (speedup 1.0000x reference)
import functools

import jax
import jax.numpy as jnp
from jax import lax
from jax.experimental import pallas as pl
from jax.experimental.pallas import tpu as pltpu
from jax.experimental.pallas import tpu_sc as plsc

CH = 128
CHM = 64
NW = 32
LANES = 16
CPT = 80
EP = CPT * NW * CH
CPTM = EP // (NW * CHM)


def _proj_tc(edge_attr, W_edge, b_edge):
    Epad, DE = edge_attr.shape
    D = W_edge.shape[1]
    BLK = 4096

    def body(ea_ref, w_ref, b_ref, out_ref):
        out_ref[...] = jnp.dot(ea_ref[...], w_ref[...],
                               preferred_element_type=jnp.float32,
                               precision=lax.Precision.HIGHEST) + b_ref[...]

    return pl.pallas_call(
        body,
        grid=(Epad // BLK,),
        in_specs=[
            pl.BlockSpec((BLK, DE), lambda i: (i, 0)),
            pl.BlockSpec((DE, D), lambda i: (0, 0)),
            pl.BlockSpec((1, D), lambda i: (0, 0)),
        ],
        out_specs=pl.BlockSpec((BLK, D), lambda i: (i, 0)),
        out_shape=jax.ShapeDtypeStruct((Epad, D), jnp.float32),
    )(edge_attr, W_edge, b_edge.reshape(1, D))


def _sc_message(x_pad, zeros_nd, proj, src_pad, dst_pad, n_real):
    NP, D = x_pad.shape
    N = n_real
    rows_per_tile = ((N + 15 * 8) // (16 * 8)) * 8
    mesh = plsc.VectorSubcoreMesh(core_axis_name="c", subcore_axis_name="s")

    @functools.partial(
        pl.kernel,
        out_type=jax.ShapeDtypeStruct((2, N, D), jnp.float32),
        mesh=mesh,
        name="sc_message",
        scratch_types=(
            [pltpu.VMEM_SHARED((NP, D), jnp.float32)]
            + [pltpu.VMEM((CHM,), jnp.int32) for _ in range(4)]
            + [pltpu.VMEM((CHM, D), jnp.float32) for _ in range(4)]
            + [pltpu.SemaphoreType.DMA for _ in range(10)]
        ),
    )
    def sc_message_k(x_hbm, zeros_hbm, proj_hbm, src_hbm, dst_hbm, out_hbm,
                     *scr):
        agg_sh = scr[0]
        sidxA, didxA, sidxB, didxB = scr[1:5]
        xrA, prA, xrB, prB = scr[5:9]
        (semA1, semA2, semB1, semB2, semGA, semGB,
         semPA, semPB, semSA, semSB) = scr[9:19]

        c = lax.axis_index("c")
        s = lax.axis_index("s")
        wid = c * 16 + s

        rb = pl.multiple_of(jnp.minimum(s * rows_per_tile, N - rows_per_tile), 8)
        pltpu.sync_copy(zeros_hbm.at[pl.ds(rb, rows_per_tile)],
                        agg_sh.at[pl.ds(rb, rows_per_tile)])

        @pl.when(s == 0)
        def _():
            pltpu.sync_copy(zeros_hbm.at[pl.ds(N, NP - N)],
                            agg_sh.at[pl.ds(N, NP - N)])
        plsc.subcore_barrier()

        def compute(xrows, prows):
            def row(r, c2):
                for j in range(D // LANES):
                    sl = pl.ds(j * LANES, LANES)
                    xrows[r, sl] = jnp.maximum(xrows[r, sl] + prows[r, sl], 0.0)
                return c2
            lax.fori_loop(0, CHM, row, None)

        def pair(it, carry):
            gba = pl.multiple_of(((2 * it) * NW + wid) * CHM, CHM)
            gbb = pl.multiple_of(((2 * it + 1) * NW + wid) * CHM, CHM)
            dA1 = pltpu.async_copy(src_hbm.at[pl.ds(gba, CHM)], sidxA, semA1)
            dA2 = pltpu.async_copy(dst_hbm.at[pl.ds(gba, CHM)], didxA, semA2)
            dB1 = pltpu.async_copy(src_hbm.at[pl.ds(gbb, CHM)], sidxB, semB1)
            dB2 = pltpu.async_copy(dst_hbm.at[pl.ds(gbb, CHM)], didxB, semB2)
            pA = pltpu.async_copy(proj_hbm.at[pl.ds(gba, CHM)], prA, semPA)
            pB = pltpu.async_copy(proj_hbm.at[pl.ds(gbb, CHM)], prB, semPB)
            dA1.wait()
            gA = pltpu.async_copy(x_hbm.at[sidxA], xrA, semGA)
            dB1.wait()
            gB = pltpu.async_copy(x_hbm.at[sidxB], xrB, semGB)
            gA.wait()
            pA.wait()
            compute(xrA, prA)
            dA2.wait()
            scA = pltpu.async_copy(xrA, agg_sh.at[didxA], semSA, add=True)
            gB.wait()
            pB.wait()
            compute(xrB, prB)
            dB2.wait()
            scB = pltpu.async_copy(xrB, agg_sh.at[didxB], semSB, add=True)
            scA.wait()
            scB.wait()
            return carry
        lax.fori_loop(0, CPTM // 2, pair, None)

        plsc.subcore_barrier()
        pltpu.sync_copy(agg_sh.at[pl.ds(rb, rows_per_tile)],
                        out_hbm.at[c, pl.ds(rb, rows_per_tile)])

    return sc_message_k(x_pad, zeros_nd, proj, src_pad, dst_pad)


def _node_tc(x, agg0, agg1, W1, b1, WaCat):
    N, D = x.shape
    BLK = 1000

    def body(x_ref, a0_ref, a1_ref, w1_ref, b1_ref, wa_ref, out_ref):
        z = x_ref[...] + a0_ref[...] + a1_ref[...]
        ne = jnp.maximum(
            jnp.dot(z, w1_ref[...], preferred_element_type=jnp.float32,
                    precision=lax.Precision.HIGHEST)
            + b1_ref[...], 0.0)
        out_ref[...] = jnp.dot(ne, wa_ref[...], preferred_element_type=jnp.float32,
                               precision=lax.Precision.HIGHEST)

    return pl.pallas_call(
        body,
        grid=(N // BLK,),
        in_specs=[
            pl.BlockSpec((BLK, D), lambda i: (i, 0)),
            pl.BlockSpec((BLK, D), lambda i: (i, 0)),
            pl.BlockSpec((BLK, D), lambda i: (i, 0)),
            pl.BlockSpec((D, D), lambda i: (0, 0)),
            pl.BlockSpec((1, D), lambda i: (0, 0)),
            pl.BlockSpec((D, D), lambda i: (0, 0)),
        ],
        out_specs=pl.BlockSpec((BLK, D), lambda i: (i, 0)),
        out_shape=jax.ShapeDtypeStruct((N, D), jnp.float32),
    )(x, agg0, agg1, W1, b1.reshape(1, D), WaCat)


def _sc_edge(P_pad, src_pad, dst_pad, ba, wb):
    NP, D2 = P_pad.shape
    H = D2 // 2
    mesh = plsc.VectorSubcoreMesh(core_axis_name="c", subcore_axis_name="s")

    @functools.partial(
        pl.kernel,
        out_type=jax.ShapeDtypeStruct((EP, LANES), jnp.float32),
        mesh=mesh,
        name="sc_edge",
        scratch_types=(
            [pltpu.VMEM((CH,), jnp.int32) for _ in range(4)]
            + [pltpu.VMEM((CH, D2), jnp.float32) for _ in range(4)]
            + [pltpu.VMEM((CH, LANES), jnp.float32) for _ in range(2)]
            + [pltpu.VMEM((H,), jnp.float32)]
            + [pltpu.VMEM((H,), jnp.float32)]
            + [pltpu.SemaphoreType.DMA for _ in range(10)]
        ),
    )
    def sc_edge_k(p_hbm, src_hbm, dst_hbm, ba_hbm, wb_hbm, out_hbm, *scr):
        sidxA, didxA, sidxB, didxB = scr[0:4]
        r1A, r2A, r1B, r2B = scr[4:8]
        accA, accB = scr[8:10]
        bav = scr[10]
        wbv = scr[11]
        (semA1, semA2, semB1, semB2, semGA1, semGA2,
         semGB1, semGB2, semOA, semOB) = scr[12:22]

        c = lax.axis_index("c")
        s = lax.axis_index("s")
        wid = c * 16 + s
        pltpu.sync_copy(ba_hbm, bav)
        pltpu.sync_copy(wb_hbm, wbv)

        def compute(r1, r2, acc):
            def row(e, c2):
                a = jnp.zeros((LANES,), jnp.float32)
                for jj in range(H // LANES):
                    sl = pl.ds(jj * LANES, LANES)
                    g = (r1[e, sl] + r2[e, pl.ds(H + jj * LANES, LANES)]
                         + bav[pl.ds(jj * LANES, LANES)])
                    a = a + jnp.maximum(g, 0.0) * wbv[pl.ds(jj * LANES, LANES)]
                acc[e, :] = a
                return c2
            lax.fori_loop(0, CH, row, None)

        def pair(it, carry):
            gba = pl.multiple_of(((2 * it) * NW + wid) * CH, CH)
            gbb = pl.multiple_of(((2 * it + 1) * NW + wid) * CH, CH)
            dA1 = pltpu.async_copy(src_hbm.at[pl.ds(gba, CH)], sidxA, semA1)
            dA2 = pltpu.async_copy(dst_hbm.at[pl.ds(gba, CH)], didxA, semA2)
            dB1 = pltpu.async_copy(src_hbm.at[pl.ds(gbb, CH)], sidxB, semB1)
            dB2 = pltpu.async_copy(dst_hbm.at[pl.ds(gbb, CH)], didxB, semB2)
            dA1.wait()
            dA2.wait()
            gA1 = pltpu.async_copy(p_hbm.at[sidxA], r1A, semGA1)
            gA2 = pltpu.async_copy(p_hbm.at[didxA], r2A, semGA2)
            dB1.wait()
            dB2.wait()
            gB1 = pltpu.async_copy(p_hbm.at[sidxB], r1B, semGB1)
            gB2 = pltpu.async_copy(p_hbm.at[didxB], r2B, semGB2)
            gA1.wait()
            gA2.wait()
            compute(r1A, r2A, accA)
            oA = pltpu.async_copy(accA, out_hbm.at[pl.ds(gba, CH)], semOA)
            gB1.wait()
            gB2.wait()
            compute(r1B, r2B, accB)
            oB = pltpu.async_copy(accB, out_hbm.at[pl.ds(gbb, CH)], semOB)
            oA.wait()
            oB.wait()
            return carry
        lax.fori_loop(0, CPT // 2, pair, None)

    return sc_edge_k(P_pad, src_pad, dst_pad, ba, wb)


def _fold_tc(acc16, bb):
    Epad = acc16.shape[0]
    R = Epad // 8
    a_r = acc16.reshape(R, 128)
    fold = jnp.zeros((128, 8), jnp.float32)
    fold = fold.at[jnp.arange(128), jnp.arange(128) // 16].set(1.0)
    BLK = 4096

    def body(a_ref, f_ref, b_ref, out_ref):
        out_ref[...] = jnp.dot(a_ref[...], f_ref[...],
                               preferred_element_type=jnp.float32,
                               precision=lax.Precision.HIGHEST) + b_ref[...]

    out = pl.pallas_call(
        body,
        grid=(R // BLK,),
        in_specs=[
            pl.BlockSpec((BLK, 128), lambda i: (i, 0)),
            pl.BlockSpec((128, 8), lambda i: (0, 0)),
            pl.BlockSpec((1, 1), lambda i: (0, 0)),
        ],
        out_specs=pl.BlockSpec((BLK, 8), lambda i: (i, 0)),
        out_shape=jax.ShapeDtypeStruct((R, 8), jnp.float32),
    )(a_r, fold, bb.reshape(1, 1))
    return out.reshape(Epad, 1)


def kernel(batch, x, edge_index, edge_attr, W_edge, b_edge, W1, b1, Wa, ba, Wb, bb):
    del batch
    N, D = x.shape
    E = edge_index.shape[1]
    src = edge_index[0]
    dst = edge_index[1]

    src_p = jnp.concatenate([src, jnp.full((EP - E,), N, src.dtype)])
    dst_p = jnp.concatenate([dst, jnp.full((EP - E,), N, dst.dtype)])
    ea_p = jnp.concatenate(
        [edge_attr, jnp.zeros((EP - E, edge_attr.shape[1]), edge_attr.dtype)])
    x_p = jnp.concatenate([x, jnp.zeros((8, D), x.dtype)])
    zeros_np = jnp.zeros_like(x_p)

    proj = _proj_tc(ea_p, W_edge, b_edge)
    aggs = _sc_message(x_p, zeros_np, proj, src_p, dst_p, N)

    WaCat = jnp.concatenate([Wa[:D], Wa[D:]], axis=1)
    P = _node_tc(x, aggs[0], aggs[1], W1, b1, WaCat)
    P_p = jnp.concatenate([P, jnp.zeros((8, P.shape[1]), P.dtype)])

    acc16 = _sc_edge(P_p, src_p, dst_p, ba, Wb.reshape(-1))
    return _fold_tc(acc16, bb)[:E]


# speedup vs baseline: 1.5784x; 1.5784x over previous
import functools

import jax
import jax.numpy as jnp
from jax import lax
from jax.experimental import pallas as pl
from jax.experimental.pallas import tpu as pltpu
from jax.experimental.pallas import tpu_sc as plsc

CH = 128
NW = 32
LANES = 16


def _proj_tc(edge_attr, W_edge, b_edge):
    E, DE = edge_attr.shape
    D = W_edge.shape[1]
    BLK = 4000

    def body(ea_ref, w_ref, b_ref, out_ref):
        out_ref[...] = jnp.dot(ea_ref[...], w_ref[...],
                               preferred_element_type=jnp.float32,
                               precision=lax.Precision.HIGHEST) + b_ref[...]

    return pl.pallas_call(
        body,
        grid=(E // BLK,),
        in_specs=[
            pl.BlockSpec((BLK, DE), lambda i: (i, 0)),
            pl.BlockSpec((DE, D), lambda i: (0, 0)),
            pl.BlockSpec((1, D), lambda i: (0, 0)),
        ],
        out_specs=pl.BlockSpec((BLK, D), lambda i: (i, 0)),
        out_shape=jax.ShapeDtypeStruct((E, D), jnp.float32),
    )(edge_attr, W_edge, b_edge.reshape(1, D))


def _sc_message(x, zeros_nd, proj, src, dst):
    N, D = x.shape
    E = src.shape[0]
    nchunk = E // CH
    cpw = (nchunk + NW - 1) // NW
    rows_per_tile = ((N + 15 * 8) // (16 * 8)) * 8
    mesh = plsc.VectorSubcoreMesh(core_axis_name="c", subcore_axis_name="s")

    @functools.partial(
        pl.kernel,
        out_type=jax.ShapeDtypeStruct((2, N, D), jnp.float32),
        mesh=mesh,
        name="sc_message",
        scratch_types=[
            pltpu.VMEM_SHARED((N, D), jnp.float32),
            pltpu.VMEM((CH,), jnp.int32),
            pltpu.VMEM((CH,), jnp.int32),
            pltpu.VMEM((CH, D), jnp.float32),
            pltpu.VMEM((CH, D), jnp.float32),
            pltpu.SemaphoreType.DMA,
        ],
    )
    def sc_message_k(x_hbm, zeros_hbm, proj_hbm, src_hbm, dst_hbm, out_hbm,
          agg_sh, sidx, didx, xrows, prows, sem):
        c = lax.axis_index("c")
        s = lax.axis_index("s")
        wid = c * 16 + s

        rb = pl.multiple_of(jnp.minimum(s * rows_per_tile, N - rows_per_tile), 8)
        pltpu.sync_copy(zeros_hbm.at[pl.ds(rb, rows_per_tile)],
                        agg_sh.at[pl.ds(rb, rows_per_tile)])
        plsc.subcore_barrier()

        def chunk(kk, carry):
            cidx = kk * NW + wid

            @pl.when(cidx < nchunk)
            def _():
                gb = cidx * CH
                pltpu.sync_copy(src_hbm.at[pl.ds(gb, CH)], sidx)
                pltpu.sync_copy(dst_hbm.at[pl.ds(gb, CH)], didx)
                gat = pltpu.async_copy(x_hbm.at[sidx], xrows, sem)
                pltpu.sync_copy(proj_hbm.at[pl.ds(gb, CH)], prows)
                gat.wait()

                def row(r, c2):
                    for j in range(D // LANES):
                        sl = pl.ds(j * LANES, LANES)
                        xrows[r, sl] = jnp.maximum(xrows[r, sl] + prows[r, sl], 0.0)
                    return c2
                lax.fori_loop(0, CH, row, None)
                pltpu.sync_copy(xrows, agg_sh.at[didx], add=True)
            return carry
        lax.fori_loop(0, cpw, chunk, None)

        plsc.subcore_barrier()
        pltpu.sync_copy(agg_sh.at[pl.ds(rb, rows_per_tile)],
                        out_hbm.at[c, pl.ds(rb, rows_per_tile)])


    return sc_message_k(x, zeros_nd, proj, src, dst)


def _node_tc(x, agg0, agg1, W1, b1, WaCat):
    N, D = x.shape
    BLK = 1000

    def body(x_ref, a0_ref, a1_ref, w1_ref, b1_ref, wa_ref, out_ref):
        z = x_ref[...] + a0_ref[...] + a1_ref[...]
        ne = jnp.maximum(
            jnp.dot(z, w1_ref[...], preferred_element_type=jnp.float32,
                    precision=lax.Precision.HIGHEST)
            + b1_ref[...], 0.0)
        out_ref[...] = jnp.dot(ne, wa_ref[...], preferred_element_type=jnp.float32,
                               precision=lax.Precision.HIGHEST)

    return pl.pallas_call(
        body,
        grid=(N // BLK,),
        in_specs=[
            pl.BlockSpec((BLK, D), lambda i: (i, 0)),
            pl.BlockSpec((BLK, D), lambda i: (i, 0)),
            pl.BlockSpec((BLK, D), lambda i: (i, 0)),
            pl.BlockSpec((D, D), lambda i: (0, 0)),
            pl.BlockSpec((1, D), lambda i: (0, 0)),
            pl.BlockSpec((D, D), lambda i: (0, 0)),
        ],
        out_specs=pl.BlockSpec((BLK, D), lambda i: (i, 0)),
        out_shape=jax.ShapeDtypeStruct((N, D), jnp.float32),
    )(x, agg0, agg1, W1, b1.reshape(1, D), WaCat)


def _sc_edge(P, src, dst, ba, wb):
    N, D2 = P.shape
    H = D2 // 2
    E = src.shape[0]
    nchunk = E // CH
    cpw = (nchunk + NW - 1) // NW
    mesh = plsc.VectorSubcoreMesh(core_axis_name="c", subcore_axis_name="s")

    @functools.partial(
        pl.kernel,
        out_type=jax.ShapeDtypeStruct((E, LANES), jnp.float32),
        mesh=mesh,
        name="sc_edge",
        scratch_types=[
            pltpu.VMEM((CH,), jnp.int32),
            pltpu.VMEM((CH,), jnp.int32),
            pltpu.VMEM((CH, D2), jnp.float32),
            pltpu.VMEM((CH, D2), jnp.float32),
            pltpu.VMEM((CH, LANES), jnp.float32),
            pltpu.VMEM((H,), jnp.float32),
            pltpu.VMEM((H,), jnp.float32),
            pltpu.SemaphoreType.DMA,
            pltpu.SemaphoreType.DMA,
        ],
    )
    def sc_edge_k(p_hbm, src_hbm, dst_hbm, ba_hbm, wb_hbm, out_hbm,
          sidx, didx, r1, r2, acc, bav, wbv, sem1, sem2):
        wid = lax.axis_index("c") * 16 + lax.axis_index("s")
        pltpu.sync_copy(ba_hbm, bav)
        pltpu.sync_copy(wb_hbm, wbv)

        def chunk(kk, carry):
            cidx = kk * NW + wid

            @pl.when(cidx < nchunk)
            def _():
                gb = cidx * CH
                pltpu.sync_copy(src_hbm.at[pl.ds(gb, CH)], sidx)
                pltpu.sync_copy(dst_hbm.at[pl.ds(gb, CH)], didx)
                g1 = pltpu.async_copy(p_hbm.at[sidx], r1, sem1)
                g2 = pltpu.async_copy(p_hbm.at[didx], r2, sem2)
                g1.wait()
                g2.wait()

                def row(e, c2):
                    a = jnp.zeros((LANES,), jnp.float32)
                    for j in range(H // LANES):
                        sl = pl.ds(j * LANES, LANES)
                        g = (r1[e, sl] + r2[e, pl.ds(H + j * LANES, LANES)]
                             + bav[pl.ds(j * LANES, LANES)])
                        a = a + jnp.maximum(g, 0.0) * wbv[pl.ds(j * LANES, LANES)]
                    acc[e, :] = a
                    return c2
                lax.fori_loop(0, CH, row, None)
                pltpu.sync_copy(acc, out_hbm.at[pl.ds(gb, CH)])
            return carry
        lax.fori_loop(0, cpw, chunk, None)

    return sc_edge_k(P, src, dst, ba, wb)


def _fold_tc(acc16, bb):
    E = acc16.shape[0]
    R = E // 8
    a_r = acc16.reshape(R, 128)
    fold = jnp.zeros((128, 8), jnp.float32)
    fold = fold.at[jnp.arange(128), jnp.arange(128) // 16].set(1.0)
    BLK = 4000

    def body(a_ref, f_ref, b_ref, out_ref):
        out_ref[...] = jnp.dot(a_ref[...], f_ref[...],
                               preferred_element_type=jnp.float32,
                               precision=lax.Precision.HIGHEST) + b_ref[...]

    out = pl.pallas_call(
        body,
        grid=(R // BLK,),
        in_specs=[
            pl.BlockSpec((BLK, 128), lambda i: (i, 0)),
            pl.BlockSpec((128, 8), lambda i: (0, 0)),
            pl.BlockSpec((1, 1), lambda i: (0, 0)),
        ],
        out_specs=pl.BlockSpec((BLK, 8), lambda i: (i, 0)),
        out_shape=jax.ShapeDtypeStruct((R, 8), jnp.float32),
    )(a_r, fold, bb.reshape(1, 1))
    return out.reshape(E, 1)


def kernel(batch, x, edge_index, edge_attr, W_edge, b_edge, W1, b1, Wa, ba, Wb, bb):
    del batch
    D = x.shape[1]
    src = edge_index[0]
    dst = edge_index[1]

    proj = _proj_tc(edge_attr, W_edge, b_edge)
    zeros_nd = jnp.zeros_like(x)
    aggs = _sc_message(x, zeros_nd, proj, src, dst)

    WaCat = jnp.concatenate([Wa[:D], Wa[D:]], axis=1)
    P = _node_tc(x, aggs[0], aggs[1], W1, b1, WaCat)

    acc16 = _sc_edge(P, src, dst, ba, Wb.reshape(-1))
    return _fold_tc(acc16, bb)
